# baseline (speedup 1.0000x reference)
import functools

import jax
import jax.numpy as jnp
import numpy as np
from jax.experimental import pallas as pl


def _count16(m01):
    a = m01
    while a.shape[1] > 128:
        h = a.shape[1] // 2
        a = a[:, :h] + a[:, h:]
    return jnp.sum(a.astype(jnp.float32), axis=1, keepdims=True)


def _count_ge(v16, t32):
    t16 = jnp.broadcast_to(t32.astype(jnp.int16), v16.shape)
    return _count16(jnp.where(v16 >= t16, np.int16(1), np.int16(0)))


def _ash_block_kernel(x_ref, wt_ref, b_ref, o_ref, *, k: int, chains: int):
    X = x_ref[...]
    RB, D = X.shape
    H = RB // chains
    kf = jnp.float32(k)
    C = range(chains)

    xs = [X[c * H:(c + 1) * H, :] for c in C]
    fills = [jnp.sum(x, axis=1, keepdims=True) * (1.0 / k) for x in xs]

    his, los = [], []
    for x in xs:
        s = jax.lax.bitcast_convert_type(x, jnp.int32)
        key = jnp.where(s < 0, s ^ jnp.int32(0x7FFFFFFF), s)
        his.append((key >> 16).astype(jnp.int16))
        los.append(key.astype(jnp.int16) ^ np.int16(-32768))

    Khi = [jnp.full((H, 1), np.int32(-32768)) for _ in C]
    for bit in range(15, -1, -1):
        ts = [Khi[c] + np.int32(1 << bit) for c in C]
        cnts = [_count_ge(his[c], ts[c]) for c in C]
        Khi = [jnp.where(cnts[c] >= kf, ts[c], Khi[c]) for c in C]

    t16s = [jnp.broadcast_to(Khi[c].astype(jnp.int16), (H, D)) for c in C]
    eq_his = [his[c] == t16s[c] for c in C]
    k2s = [kf - _count16(jnp.where(his[c] > t16s[c], np.int16(1),
                                   np.int16(0))) for c in C]
    lo_ms = [jnp.where(eq_his[c], los[c], np.int16(-32768)) for c in C]
    Klo = [jnp.full((H, 1), np.int32(-32768)) for _ in C]
    for bit in range(15, -1, -1):
        ts = [Klo[c] + np.int32(1 << bit) for c in C]
        cnts = [_count_ge(lo_ms[c], ts[c]) for c in C]
        Klo = [jnp.where(cnts[c] >= k2s[c], ts[c], Klo[c]) for c in C]

    outs = []
    for c in C:
        hi, lo_m, eq_hi, t16 = his[c], lo_ms[c], eq_his[c], t16s[c]
        Klo16 = jnp.broadcast_to(Klo[c].astype(jnp.int16), (H, D))
        gt = (hi > t16) | (eq_hi & (lo_m > Klo16))
        eq = eq_hi & (lo_m == Klo16)
        cnt_gt = _count16(jnp.where(gt, np.int16(1), np.int16(0)))
        cnt_eq = _count16(jnp.where(eq, np.int16(1), np.int16(0)))
        need = kf - cnt_gt

        excess_any = jnp.max(cnt_eq - need) > 0.5

        def slow_path(_, gt=gt, eq=eq, cnt_eq=cnt_eq, need=need):
            idx = jax.lax.broadcasted_iota(jnp.int16, (H, D), 1)
            idx_m = jnp.where(eq, idx, np.int16(-1))
            m = cnt_eq - need + 1.0
            J = jnp.zeros((H, 1), jnp.int32)
            for bit in range(10, -1, -1):
                t = J | np.int32(1 << bit)
                cnt = _count_ge(idx_m, t)
                J = jnp.where(cnt >= m, t, J)
            J16 = jnp.broadcast_to(J.astype(jnp.int16), (H, D))
            return jnp.where(gt | (eq & (idx <= J16)),
                             np.int16(1), np.int16(0))

        def fast_path(_, gt=gt, eq=eq):
            return jnp.where(gt | eq, np.int16(1), np.int16(0))

        mask01 = jax.lax.cond(excess_any, slow_path, fast_path, operand=None)
        maskf = mask01.astype(jnp.float32)

        colsum = jnp.dot(maskf, wt_ref[...],
                         preferred_element_type=jnp.float32)
        logits = fills[c] * colsum + b_ref[...]
        mx = jnp.max(logits, axis=1, keepdims=True)
        lse = mx + jnp.log(jnp.sum(jnp.exp(logits - mx), axis=1,
                                   keepdims=True))
        outs.append(-lse)

    o_ref[...] = jnp.concatenate(outs, axis=0) if chains > 1 else outs[0]


@jax.jit
def kernel(features, logits, W_fc, b_fc):
    del logits
    B, D = features.shape
    C = W_fc.shape[0]
    k = D - int(round(D * 90 / 100.0))

    R = min(512, B)
    chains = 2 if R >= 512 else 1
    wt = W_fc.T
    b2 = b_fc.reshape(1, C)

    out = pl.pallas_call(
        functools.partial(_ash_block_kernel, k=k, chains=chains),
        grid=(B // R,),
        in_specs=[
            pl.BlockSpec((R, D), lambda i: (i, 0)),
            pl.BlockSpec((D, C), lambda i: (0, 0)),
            pl.BlockSpec((1, C), lambda i: (0, 0)),
        ],
        out_specs=pl.BlockSpec((R, 1), lambda i: (i, 0)),
        out_shape=jax.ShapeDtypeStruct((B, 1), jnp.float32),
    )(features, wt, b2)
    return out


# speedup vs baseline: 1.0344x; 1.0344x over previous
import functools

import jax
import jax.numpy as jnp
import numpy as np
from jax.experimental import pallas as pl


def _tree128(a):
    while a.shape[1] > 128:
        h = a.shape[1] // 2
        a = a[:, :h] + a[:, h:]
    return a


def _count16(m01):
    return jnp.sum(_tree128(m01).astype(jnp.float32), axis=1, keepdims=True)


def _count_ge(v16, t32):
    t16 = jnp.broadcast_to(t32.astype(jnp.int16), v16.shape)
    return _count16(jnp.where(v16 >= t16, np.int16(1), np.int16(0)))


def _ash_block_kernel(x_ref, wt_ref, b_ref, ones_ref, o_ref, *, k: int):
    x = x_ref[...]
    R, D = x.shape
    kf = jnp.float32(k)

    fill = jnp.dot(x, ones_ref[...],
                   preferred_element_type=jnp.float32)[:, :1] * (1.0 / k)

    s = jax.lax.bitcast_convert_type(x, jnp.int32)
    key = jnp.where(s < 0, s ^ jnp.int32(0x7FFFFFFF), s)

    hi = (key >> 16).astype(jnp.int16)
    lo = key.astype(jnp.int16) ^ np.int16(-32768)

    Khi = jnp.full((R, 1), np.int32(-32768))
    for bit in range(15, -1, -1):
        t = Khi + np.int32(1 << bit)
        cnt = _count_ge(hi, t)
        Khi = jnp.where(cnt >= kf, t, Khi)

    t16 = jnp.broadcast_to(Khi.astype(jnp.int16), (R, D))
    eq_hi = hi == t16
    cnt_gt_hi = _count16(jnp.where(hi > t16, np.int16(1), np.int16(0)))
    k2 = kf - cnt_gt_hi
    lo_m = jnp.where(eq_hi, lo, np.int16(-32768))
    Klo = jnp.full((R, 1), np.int32(-32768))
    for bit in range(15, -1, -1):
        t = Klo + np.int32(1 << bit)
        cnt = _count_ge(lo_m, t)
        Klo = jnp.where(cnt >= k2, t, Klo)

    Klo16 = jnp.broadcast_to(Klo.astype(jnp.int16), (R, D))
    ge = (hi > t16) | (eq_hi & (lo_m >= Klo16))
    mask_bf = jnp.where(ge, jnp.bfloat16(1), jnp.bfloat16(0))
    cnt_ge = jnp.sum(_tree128(mask_bf).astype(jnp.float32), axis=1,
                     keepdims=True)

    excess_any = jnp.max(cnt_ge) > kf + 0.5

    def slow_path(_):
        eq = eq_hi & (lo_m == Klo16)
        gt = (hi > t16) | (eq_hi & (lo_m > Klo16))
        cnt_eq = _count16(jnp.where(eq, np.int16(1), np.int16(0)))
        cnt_gt = cnt_ge - cnt_eq
        need = kf - cnt_gt
        m = cnt_eq - need + 1.0
        idx = jax.lax.broadcasted_iota(jnp.int16, (R, D), 1)
        idx_m = jnp.where(eq, idx, np.int16(-1))
        J = jnp.zeros((R, 1), jnp.int32)
        for bit in range(10, -1, -1):
            t = J | np.int32(1 << bit)
            cnt = _count_ge(idx_m, t)
            J = jnp.where(cnt >= m, t, J)
        J16 = jnp.broadcast_to(J.astype(jnp.int16), (R, D))
        return jnp.where(gt | (eq & (idx <= J16)),
                         jnp.bfloat16(1), jnp.bfloat16(0))

    def fast_path(_):
        return mask_bf

    mask = jax.lax.cond(excess_any, slow_path, fast_path, operand=None)

    colsum = jnp.dot(mask, wt_ref[...],
                     preferred_element_type=jnp.float32)
    logits = fill * colsum + b_ref[...]
    mx = jnp.max(logits, axis=1, keepdims=True)
    lse = mx + jnp.log(jnp.sum(jnp.exp(logits - mx), axis=1, keepdims=True))
    o_ref[...] = -lse


@jax.jit
def kernel(features, logits, W_fc, b_fc):
    del logits
    B, D = features.shape
    C = W_fc.shape[0]
    k = D - int(round(D * 90 / 100.0))

    R = min(256, B)
    wt = W_fc.T.astype(jnp.bfloat16)
    b2 = b_fc.reshape(1, C)
    ones = jnp.ones((D, 8), jnp.float32)

    out = pl.pallas_call(
        functools.partial(_ash_block_kernel, k=k),
        grid=(B // R,),
        in_specs=[
            pl.BlockSpec((R, D), lambda i: (i, 0)),
            pl.BlockSpec((D, C), lambda i: (0, 0)),
            pl.BlockSpec((1, C), lambda i: (0, 0)),
            pl.BlockSpec((D, 8), lambda i: (0, 0)),
        ],
        out_specs=pl.BlockSpec((R, 1), lambda i: (i, 0)),
        out_shape=jax.ShapeDtypeStruct((B, 1), jnp.float32),
    )(features, wt, b2, ones)
    return out
